# Initial kernel scaffold; baseline (speedup 1.0000x reference)
import jax
import jax.numpy as jnp
from jax.experimental import pallas as pl


def kernel(x, edge_index, Ws):
    raise NotImplementedError("write your pallas kernel here")


# speedup vs baseline: 416.2954x; 416.2954x over previous
import functools

import jax
import jax.numpy as jnp
from jax import lax
from jax.experimental import pallas as pl
from jax.experimental.pallas import tpu as pltpu
from jax.experimental.pallas import tpu_sc as plsc

NN = 100000
NE = 6400000
NBLK = 12
NC = 2
NS = 16
NW = NC * NS
LANES = 16
NP = 100096
SLICE = NP // NS
EPW = NE // NW
CHUNK = 10000
NCHUNK = EPW // CHUNK

_MESH = plsc.VectorSubcoreMesh(
    core_axis_name="c", subcore_axis_name="s", num_cores=NC, num_subcores=NS
)


def _fill(ref, n, value):
    def body(i, carry):
        ref[pl.ds(i * LANES, LANES)] = jnp.full((LANES,), value, ref.dtype)
        return carry
    lax.fori_loop(0, n // LANES, body, 0)


@functools.partial(
    pl.kernel,
    out_type=jax.ShapeDtypeStruct((NC * NP,), jnp.float32),
    mesh=_MESH,
    scratch_types=[
        pltpu.VMEM_SHARED((NP,), jnp.float32),
        pltpu.VMEM((CHUNK,), jnp.int32),
        pltpu.VMEM((CHUNK,), jnp.float32),
        pltpu.VMEM((SLICE,), jnp.float32),
    ],
)
def _deg_kernel(dst_hbm, out_hbm, acc_sp, idx_v, ones_v, zbuf):
    c = lax.axis_index("c")
    s = lax.axis_index("s")
    nb = s * SLICE
    _fill(zbuf, SLICE, 0.0)
    pltpu.sync_copy(zbuf, acc_sp.at[pl.ds(nb, SLICE)])
    _fill(ones_v, CHUNK, 1.0)
    plsc.subcore_barrier()
    ebase = (c * NS + s) * EPW

    def chunk(k, carry):
        off = ebase + k * CHUNK
        pltpu.sync_copy(dst_hbm.at[pl.ds(off, CHUNK)], idx_v)
        pltpu.sync_copy(ones_v, acc_sp.at[idx_v], add=True)
        return carry

    lax.fori_loop(0, NCHUNK, chunk, 0)
    plsc.subcore_barrier()
    pltpu.sync_copy(acc_sp.at[pl.ds(nb, SLICE)], zbuf)
    pltpu.sync_copy(zbuf, out_hbm.at[pl.ds(c * NP + nb, SLICE)])


@functools.partial(
    pl.kernel,
    out_type=jax.ShapeDtypeStruct((NC * NP,), jnp.float32),
    mesh=_MESH,
    scratch_types=[
        pltpu.VMEM_SHARED((NP,), jnp.float32),
        pltpu.VMEM_SHARED((NP,), jnp.float32),
        pltpu.VMEM((SLICE,), jnp.float32),
        pltpu.VMEM((SLICE,), jnp.float32),
        pltpu.VMEM((SLICE,), jnp.float32),
        pltpu.VMEM((CHUNK,), jnp.int32),
        pltpu.VMEM((CHUNK,), jnp.int32),
        pltpu.VMEM((CHUNK,), jnp.float32),
    ],
)
def _spmv_kernel(a_hbm, b_hbm, scale_hbm, src_hbm, dst_hbm, out_hbm,
                 u_sp, acc_sp, av, bv, sv, src_v, dst_v, vals_v):
    c = lax.axis_index("c")
    s = lax.axis_index("s")
    nb = s * SLICE
    pltpu.sync_copy(a_hbm.at[pl.ds(nb, SLICE)], av)
    pltpu.sync_copy(b_hbm.at[pl.ds(nb, SLICE)], bv)
    pltpu.sync_copy(scale_hbm.at[pl.ds(nb, SLICE)], sv)

    def ubody(i, carry):
        d = pl.ds(i * LANES, LANES)
        sv[d] = sv[d] * (av[d] + bv[d])
        return carry

    lax.fori_loop(0, SLICE // LANES, ubody, 0)
    pltpu.sync_copy(sv, u_sp.at[pl.ds(nb, SLICE)])
    _fill(av, SLICE, 0.0)
    pltpu.sync_copy(av, acc_sp.at[pl.ds(nb, SLICE)])
    plsc.subcore_barrier()

    ebase = (c * NS + s) * EPW

    def chunk(k, carry):
        off = ebase + k * CHUNK
        pltpu.sync_copy(src_hbm.at[pl.ds(off, CHUNK)], src_v)
        pltpu.sync_copy(u_sp.at[src_v], vals_v)
        pltpu.sync_copy(dst_hbm.at[pl.ds(off, CHUNK)], dst_v)
        pltpu.sync_copy(vals_v, acc_sp.at[dst_v], add=True)
        return carry

    lax.fori_loop(0, NCHUNK, chunk, 0)
    plsc.subcore_barrier()
    pltpu.sync_copy(acc_sp.at[pl.ds(nb, SLICE)], av)
    pltpu.sync_copy(av, out_hbm.at[pl.ds(c * NP + nb, SLICE)])


def kernel(x, edge_index, Ws):
    del Ws
    src = edge_index[0]
    dst = edge_index[1]
    xcol = x[:, 3].astype(jnp.float32)
    xp = jnp.zeros((NP,), jnp.float32).at[:NN].set(xcol)

    degp = _deg_kernel(dst)
    deg = degp[:NP] + degp[NP:]
    dis = jnp.where(deg > 0, lax.rsqrt(jnp.maximum(deg, 1e-12)), 0.0)
    inv = dis * dis

    a, b, scale = xp, jnp.zeros((NP,), jnp.float32), dis
    for _ in range(NBLK):
        p = _spmv_kernel(a, b, scale, src, dst)
        a, b, scale = p[:NP], p[NP:], inv
    h = dis * (a + b)
    return h[:NN].reshape(NN, 1, 1)


# speedup vs baseline: 527.6566x; 1.2675x over previous
import functools

import jax
import jax.numpy as jnp
from jax import lax
from jax.experimental import pallas as pl
from jax.experimental.pallas import tpu as pltpu
from jax.experimental.pallas import tpu_sc as plsc

NN = 100000
NE = 6400000
NBLK = 12
NC = 2
NS = 16
NW = NC * NS
LANES = 16
NP = 100096
SLICE = NP // NS
EPW = NE // NW
CHUNK = 20000
NCHUNK = EPW // CHUNK
NBUF = 2

_MESH = plsc.VectorSubcoreMesh(
    core_axis_name="c", subcore_axis_name="s", num_cores=NC, num_subcores=NS
)


def _fill(ref, n, value):
    def body(i, carry):
        ref[pl.ds(i * LANES, LANES)] = jnp.full((LANES,), value, ref.dtype)
        return carry
    lax.fori_loop(0, n // LANES, body, 0)


@functools.partial(
    pl.kernel,
    out_type=jax.ShapeDtypeStruct((NC * NP,), jnp.float32),
    mesh=_MESH,
    scratch_types=[
        pltpu.VMEM_SHARED((NP,), jnp.float32),
        pltpu.VMEM((CHUNK,), jnp.int32),
        pltpu.VMEM((CHUNK,), jnp.int32),
        pltpu.VMEM((CHUNK,), jnp.float32),
        pltpu.VMEM((SLICE,), jnp.float32),
        pltpu.SemaphoreType.DMA((NBUF,)),
    ],
)
def _deg_kernel(dst_hbm, out_hbm, acc_sp, idx_r0, idx_r1, ones_v, zbuf, sem):
    idx2 = (idx_r0, idx_r1)
    c = lax.axis_index("c")
    s = lax.axis_index("s")
    nb = s * SLICE
    _fill(zbuf, SLICE, 0.0)
    pltpu.sync_copy(zbuf, acc_sp.at[pl.ds(nb, SLICE)])
    _fill(ones_v, CHUNK, 1.0)
    plsc.subcore_barrier()
    ebase = (c * NS + s) * EPW

    for b in range(NBUF):
        pltpu.async_copy(
            dst_hbm.at[pl.ds(ebase + b * CHUNK, CHUNK)], idx2[b], sem.at[b])

    def group(g, carry):
        for b in range(NBUF):
            k = g * NBUF + b
            pltpu.make_async_copy(
                dst_hbm.at[pl.ds(ebase, CHUNK)], idx2[b], sem.at[b]).wait()
            pltpu.sync_copy(ones_v, acc_sp.at[idx2[b]], add=True)

            @pl.when(k + NBUF < NCHUNK)
            def _():
                off = ebase + (k + NBUF) * CHUNK
                pltpu.async_copy(
                    dst_hbm.at[pl.ds(off, CHUNK)], idx2[b], sem.at[b])
        return carry

    lax.fori_loop(0, NCHUNK // NBUF, group, 0)
    plsc.subcore_barrier()
    pltpu.sync_copy(acc_sp.at[pl.ds(nb, SLICE)], zbuf)
    pltpu.sync_copy(zbuf, out_hbm.at[pl.ds(c * NP + nb, SLICE)])


@functools.partial(
    pl.kernel,
    out_type=jax.ShapeDtypeStruct((NC * NP,), jnp.float32),
    mesh=_MESH,
    scratch_types=[
        pltpu.VMEM_SHARED((NP,), jnp.float32),
        pltpu.VMEM_SHARED((NP,), jnp.float32),
        pltpu.VMEM((SLICE,), jnp.float32),
        pltpu.VMEM((SLICE,), jnp.float32),
        pltpu.VMEM((CHUNK,), jnp.int32),
        pltpu.VMEM((CHUNK,), jnp.int32),
        pltpu.VMEM((CHUNK,), jnp.int32),
        pltpu.VMEM((CHUNK,), jnp.int32),
        pltpu.VMEM((CHUNK,), jnp.float32),
        pltpu.SemaphoreType.DMA((NBUF,)),
        pltpu.SemaphoreType.DMA((NBUF,)),
    ],
)
def _spmv_kernel(a_hbm, b_hbm, scale_hbm, src_hbm, dst_hbm, out_hbm,
                 u_sp, acc_sp, av, bv, src_r0, src_r1, dst_r0, dst_r1,
                 vals_v, sem_s, sem_d):
    src2 = (src_r0, src_r1)
    dst2 = (dst_r0, dst_r1)
    c = lax.axis_index("c")
    s = lax.axis_index("s")
    nb = s * SLICE
    sv = vals_v.at[pl.ds(0, SLICE)]
    pltpu.sync_copy(a_hbm.at[pl.ds(nb, SLICE)], av)
    pltpu.sync_copy(b_hbm.at[pl.ds(nb, SLICE)], bv)
    pltpu.sync_copy(scale_hbm.at[pl.ds(nb, SLICE)], sv)

    def ubody(i, carry):
        d = pl.ds(i * LANES, LANES)
        vals_v[d] = vals_v[d] * (av[d] + bv[d])
        return carry

    lax.fori_loop(0, SLICE // LANES, ubody, 0)
    pltpu.sync_copy(sv, u_sp.at[pl.ds(nb, SLICE)])
    _fill(av, SLICE, 0.0)
    pltpu.sync_copy(av, acc_sp.at[pl.ds(nb, SLICE)])
    plsc.subcore_barrier()

    ebase = (c * NS + s) * EPW

    for b in range(NBUF):
        off = ebase + b * CHUNK
        pltpu.async_copy(src_hbm.at[pl.ds(off, CHUNK)], src2[b], sem_s.at[b])
        pltpu.async_copy(dst_hbm.at[pl.ds(off, CHUNK)], dst2[b], sem_d.at[b])

    def group(g, carry):
        for b in range(NBUF):
            k = g * NBUF + b
            pltpu.make_async_copy(
                src_hbm.at[pl.ds(ebase, CHUNK)], src2[b], sem_s.at[b]).wait()
            pltpu.sync_copy(u_sp.at[src2[b]], vals_v)
            pltpu.make_async_copy(
                dst_hbm.at[pl.ds(ebase, CHUNK)], dst2[b], sem_d.at[b]).wait()
            pltpu.sync_copy(vals_v, acc_sp.at[dst2[b]], add=True)

            @pl.when(k + NBUF < NCHUNK)
            def _():
                off = ebase + (k + NBUF) * CHUNK
                pltpu.async_copy(
                    src_hbm.at[pl.ds(off, CHUNK)], src2[b], sem_s.at[b])
                pltpu.async_copy(
                    dst_hbm.at[pl.ds(off, CHUNK)], dst2[b], sem_d.at[b])
        return carry

    lax.fori_loop(0, NCHUNK // NBUF, group, 0)
    plsc.subcore_barrier()
    pltpu.sync_copy(acc_sp.at[pl.ds(nb, SLICE)], av)
    pltpu.sync_copy(av, out_hbm.at[pl.ds(c * NP + nb, SLICE)])


def kernel(x, edge_index, Ws):
    del Ws
    src = edge_index[0]
    dst = edge_index[1]
    xcol = x[:, 3].astype(jnp.float32)
    xp = jnp.zeros((NP,), jnp.float32).at[:NN].set(xcol)

    degp = _deg_kernel(dst)
    deg = degp[:NP] + degp[NP:]
    dis = jnp.where(deg > 0, lax.rsqrt(jnp.maximum(deg, 1e-12)), 0.0)
    inv = dis * dis

    a, b, scale = xp, jnp.zeros((NP,), jnp.float32), dis
    for _ in range(NBLK):
        p = _spmv_kernel(a, b, scale, src, dst)
        a, b, scale = p[:NP], p[NP:], inv
    h = dis * (a + b)
    return h[:NN].reshape(NN, 1, 1)


# speedup vs baseline: 540.2383x; 1.0238x over previous
import functools

import jax
import jax.numpy as jnp
from jax import lax
from jax.experimental import pallas as pl
from jax.experimental.pallas import tpu as pltpu
from jax.experimental.pallas import tpu_sc as plsc

NN = 100000
NE = 6400000
NBLK = 12
NC = 2
NS = 16
NW = NC * NS
LANES = 16
NP = 100096
SLICE = NP // NS
EPW = NE // NW
CHUNK = 10000
NCHUNK = EPW // CHUNK
NBUF = 2
NIDX = 3

_MESH = plsc.VectorSubcoreMesh(
    core_axis_name="c", subcore_axis_name="s", num_cores=NC, num_subcores=NS
)


def _fill(ref, n, value):
    def body(i, carry):
        ref[pl.ds(i * LANES, LANES)] = jnp.full((LANES,), value, ref.dtype)
        return carry
    lax.fori_loop(0, n // LANES, body, 0)


@functools.partial(
    pl.kernel,
    out_type=jax.ShapeDtypeStruct((NC * NP,), jnp.float32),
    mesh=_MESH,
    scratch_types=[
        pltpu.VMEM_SHARED((NP,), jnp.float32),
        pltpu.VMEM((CHUNK,), jnp.int32),
        pltpu.VMEM((CHUNK,), jnp.int32),
        pltpu.VMEM((CHUNK,), jnp.float32),
        pltpu.VMEM((SLICE,), jnp.float32),
        pltpu.SemaphoreType.DMA((NBUF,)),
    ],
)
def _deg_kernel(dst_hbm, out_hbm, acc_sp, idx_r0, idx_r1, ones_v, zbuf, sem):
    idx2 = (idx_r0, idx_r1)
    c = lax.axis_index("c")
    s = lax.axis_index("s")
    nb = s * SLICE
    _fill(zbuf, SLICE, 0.0)
    pltpu.sync_copy(zbuf, acc_sp.at[pl.ds(nb, SLICE)])
    _fill(ones_v, CHUNK, 1.0)
    plsc.subcore_barrier()
    ebase = (c * NS + s) * EPW

    for b in range(NBUF):
        pltpu.async_copy(
            dst_hbm.at[pl.ds(ebase + b * CHUNK, CHUNK)], idx2[b], sem.at[b])

    def group(g, carry):
        for b in range(NBUF):
            k = g * NBUF + b
            pltpu.make_async_copy(
                dst_hbm.at[pl.ds(ebase, CHUNK)], idx2[b], sem.at[b]).wait()
            pltpu.sync_copy(ones_v, acc_sp.at[idx2[b]], add=True)

            @pl.when(k + NBUF < NCHUNK)
            def _():
                off = ebase + (k + NBUF) * CHUNK
                pltpu.async_copy(
                    dst_hbm.at[pl.ds(off, CHUNK)], idx2[b], sem.at[b])
        return carry

    lax.fori_loop(0, NCHUNK // NBUF, group, 0)
    plsc.subcore_barrier()
    pltpu.sync_copy(acc_sp.at[pl.ds(nb, SLICE)], zbuf)
    pltpu.sync_copy(zbuf, out_hbm.at[pl.ds(c * NP + nb, SLICE)])


@functools.partial(
    pl.kernel,
    out_type=jax.ShapeDtypeStruct((NC * NP,), jnp.float32),
    mesh=_MESH,
    scratch_types=[
        pltpu.VMEM_SHARED((NP,), jnp.float32),
        pltpu.VMEM_SHARED((NP,), jnp.float32),
        pltpu.VMEM((SLICE,), jnp.float32),
        pltpu.VMEM((SLICE,), jnp.float32),
        pltpu.VMEM((CHUNK,), jnp.int32),
        pltpu.VMEM((CHUNK,), jnp.int32),
        pltpu.VMEM((CHUNK,), jnp.int32),
        pltpu.VMEM((CHUNK,), jnp.int32),
        pltpu.VMEM((CHUNK,), jnp.int32),
        pltpu.VMEM((CHUNK,), jnp.int32),
        pltpu.VMEM((CHUNK,), jnp.float32),
        pltpu.VMEM((CHUNK,), jnp.float32),
        pltpu.SemaphoreType.DMA((NIDX,)),
        pltpu.SemaphoreType.DMA((NIDX,)),
        pltpu.SemaphoreType.DMA((2,)),
        pltpu.SemaphoreType.DMA((2,)),
    ],
)
def _spmv_kernel(a_hbm, b_hbm, scale_hbm, src_hbm, dst_hbm, out_hbm,
                 u_sp, acc_sp, av, bv, src_r0, src_r1, src_r2,
                 dst_r0, dst_r1, dst_r2, vals_0, vals_1,
                 sem_s, sem_d, sem_g, sem_a):
    srcs = (src_r0, src_r1, src_r2)
    dsts = (dst_r0, dst_r1, dst_r2)
    vals = (vals_0, vals_1)
    c = lax.axis_index("c")
    s = lax.axis_index("s")
    nb = s * SLICE
    sv = vals_0.at[pl.ds(0, SLICE)]
    pltpu.sync_copy(a_hbm.at[pl.ds(nb, SLICE)], av)
    pltpu.sync_copy(b_hbm.at[pl.ds(nb, SLICE)], bv)
    pltpu.sync_copy(scale_hbm.at[pl.ds(nb, SLICE)], sv)

    def ubody(i, carry):
        d = pl.ds(i * LANES, LANES)
        vals_0[d] = vals_0[d] * (av[d] + bv[d])
        return carry

    lax.fori_loop(0, SLICE // LANES, ubody, 0)
    pltpu.sync_copy(sv, u_sp.at[pl.ds(nb, SLICE)])
    _fill(av, SLICE, 0.0)
    pltpu.sync_copy(av, acc_sp.at[pl.ds(nb, SLICE)])
    plsc.subcore_barrier()

    ebase = (c * NS + s) * EPW

    def _wait_src(k):
        pltpu.make_async_copy(
            src_hbm.at[pl.ds(ebase, CHUNK)], srcs[k % NIDX],
            sem_s.at[k % NIDX]).wait()

    def _wait_dst(k):
        pltpu.make_async_copy(
            dst_hbm.at[pl.ds(ebase, CHUNK)], dsts[k % NIDX],
            sem_d.at[k % NIDX]).wait()

    def _wait_gather(k):
        pltpu.make_async_copy(
            u_sp.at[srcs[k % NIDX]], vals[k % 2], sem_g.at[k % 2]).wait()

    def _wait_scatter(k):
        pltpu.make_async_copy(
            vals[k % 2], acc_sp.at[dsts[k % NIDX]], sem_a.at[k % 2]).wait()

    for k in range(NIDX):
        off = ebase + k * CHUNK
        pltpu.async_copy(src_hbm.at[pl.ds(off, CHUNK)], srcs[k % NIDX],
                         sem_s.at[k % NIDX])
        pltpu.async_copy(dst_hbm.at[pl.ds(off, CHUNK)], dsts[k % NIDX],
                         sem_d.at[k % NIDX])

    for k in range(NCHUNK):
        if k >= 1:
            _wait_gather(k - 1)
            _wait_dst(k - 1)
            pltpu.async_copy(vals[(k - 1) % 2], acc_sp.at[dsts[(k - 1) % NIDX]],
                             sem_a.at[(k - 1) % 2], add=True)
            if k + 2 < NCHUNK:
                off = ebase + (k + 2) * CHUNK
                pltpu.async_copy(src_hbm.at[pl.ds(off, CHUNK)],
                                 srcs[(k + 2) % NIDX], sem_s.at[(k + 2) % NIDX])
        if k >= 2:
            _wait_scatter(k - 2)
            if k + 1 < NCHUNK:
                off = ebase + (k + 1) * CHUNK
                pltpu.async_copy(dst_hbm.at[pl.ds(off, CHUNK)],
                                 dsts[(k + 1) % NIDX], sem_d.at[(k + 1) % NIDX])
        _wait_src(k)
        pltpu.async_copy(u_sp.at[srcs[k % NIDX]], vals[k % 2],
                         sem_g.at[k % 2])

    _wait_gather(NCHUNK - 1)
    _wait_dst(NCHUNK - 1)
    pltpu.async_copy(vals[(NCHUNK - 1) % 2],
                     acc_sp.at[dsts[(NCHUNK - 1) % NIDX]],
                     sem_a.at[(NCHUNK - 1) % 2], add=True)
    _wait_scatter(NCHUNK - 2)
    _wait_scatter(NCHUNK - 1)
    plsc.subcore_barrier()
    pltpu.sync_copy(acc_sp.at[pl.ds(nb, SLICE)], av)
    pltpu.sync_copy(av, out_hbm.at[pl.ds(c * NP + nb, SLICE)])


def kernel(x, edge_index, Ws):
    del Ws
    src = edge_index[0]
    dst = edge_index[1]
    xcol = x[:, 3].astype(jnp.float32)
    xp = jnp.zeros((NP,), jnp.float32).at[:NN].set(xcol)

    degp = _deg_kernel(dst)
    deg = degp[:NP] + degp[NP:]
    dis = jnp.where(deg > 0, lax.rsqrt(jnp.maximum(deg, 1e-12)), 0.0)
    inv = dis * dis

    a, b, scale = xp, jnp.zeros((NP,), jnp.float32), dis
    for _ in range(NBLK):
        p = _spmv_kernel(a, b, scale, src, dst)
        a, b, scale = p[:NP], p[NP:], inv
    h = dis * (a + b)
    return h[:NN].reshape(NN, 1, 1)


# speedup vs baseline: 701.5136x; 1.2985x over previous
import functools

import jax
import jax.numpy as jnp
from jax import lax
from jax.experimental import pallas as pl
from jax.experimental.pallas import tpu as pltpu
from jax.experimental.pallas import tpu_sc as plsc

NN = 100000
NE = 6400000
NBLK = 12
NC = 2
NS = 16
NW = NC * NS
LANES = 16
NP = 100096
SLICE = NP // NS
EPW = NE // NW
CHUNK = 10000
NCHUNK = EPW // CHUNK
NBUF = 2
CH = 2000
NCH = EPW // CH
NIDX = 3

_MESH = plsc.VectorSubcoreMesh(
    core_axis_name="c", subcore_axis_name="s", num_cores=NC, num_subcores=NS
)


def _fill(ref, n, value):
    def body(i, carry):
        ref[pl.ds(i * LANES, LANES)] = jnp.full((LANES,), value, ref.dtype)
        return carry
    lax.fori_loop(0, n // LANES, body, 0)


@functools.partial(
    pl.kernel,
    out_type=jax.ShapeDtypeStruct((NC * NP,), jnp.float32),
    mesh=_MESH,
    scratch_types=[
        pltpu.VMEM_SHARED((NP,), jnp.float32),
        pltpu.VMEM((CHUNK,), jnp.int32),
        pltpu.VMEM((CHUNK,), jnp.int32),
        pltpu.VMEM((CHUNK,), jnp.float32),
        pltpu.VMEM((SLICE,), jnp.float32),
        pltpu.SemaphoreType.DMA((NBUF,)),
    ],
)
def _deg_kernel(dst_hbm, out_hbm, acc_sp, idx_r0, idx_r1, ones_v, zbuf, sem):
    idx2 = (idx_r0, idx_r1)
    c = lax.axis_index("c")
    s = lax.axis_index("s")
    nb = s * SLICE
    _fill(zbuf, SLICE, 0.0)
    pltpu.sync_copy(zbuf, acc_sp.at[pl.ds(nb, SLICE)])
    _fill(ones_v, CHUNK, 1.0)
    plsc.subcore_barrier()
    ebase = (c * NS + s) * EPW

    for b in range(NBUF):
        pltpu.async_copy(
            dst_hbm.at[pl.ds(ebase + b * CHUNK, CHUNK)], idx2[b], sem.at[b])

    def group(g, carry):
        for b in range(NBUF):
            k = g * NBUF + b
            pltpu.make_async_copy(
                dst_hbm.at[pl.ds(ebase, CHUNK)], idx2[b], sem.at[b]).wait()
            pltpu.sync_copy(ones_v, acc_sp.at[idx2[b]], add=True)

            @pl.when(k + NBUF < NCHUNK)
            def _():
                off = ebase + (k + NBUF) * CHUNK
                pltpu.async_copy(
                    dst_hbm.at[pl.ds(off, CHUNK)], idx2[b], sem.at[b])
        return carry

    lax.fori_loop(0, NCHUNK // NBUF, group, 0)
    plsc.subcore_barrier()
    pltpu.sync_copy(acc_sp.at[pl.ds(nb, SLICE)], zbuf)
    pltpu.sync_copy(zbuf, out_hbm.at[pl.ds(c * NP + nb, SLICE)])


@functools.partial(
    pl.kernel,
    out_type=jax.ShapeDtypeStruct((NC * NP,), jnp.float32),
    mesh=_MESH,
    compiler_params=pltpu.CompilerParams(needs_layout_passes=False),
    scratch_types=[
        pltpu.VMEM_SHARED((NP,), jnp.float32),
        pltpu.VMEM((NP,), jnp.float32),
        pltpu.VMEM((CH,), jnp.int32),
        pltpu.VMEM((CH,), jnp.int32),
        pltpu.VMEM((CH,), jnp.int32),
        pltpu.VMEM((CH,), jnp.int32),
        pltpu.VMEM((CH,), jnp.int32),
        pltpu.VMEM((CH,), jnp.float32),
        pltpu.VMEM((CH,), jnp.float32),
        pltpu.SemaphoreType.DMA((2,)),
        pltpu.SemaphoreType.DMA((NIDX,)),
        pltpu.SemaphoreType.DMA((1,)),
        pltpu.SemaphoreType.DMA((2,)),
    ],
)
def _spmv_kernel(u_hbm, src_hbm, dst_hbm, out_hbm,
                 acc_sp, u_loc, src_r0, src_r1, dst_r0, dst_r1, dst_r2,
                 vals_0, vals_1, sem_s, sem_d, sem_u, sem_a):
    srcs = (src_r0, src_r1)
    dsts = (dst_r0, dst_r1, dst_r2)
    vals = (vals_0, vals_1)
    c = lax.axis_index("c")
    s = lax.axis_index("s")
    nb = s * SLICE
    ebase = (c * NS + s) * EPW

    pltpu.async_copy(u_hbm, u_loc, sem_u.at[0])
    for k in range(2):
        pltpu.async_copy(src_hbm.at[pl.ds(ebase + k * CH, CH)], srcs[k],
                         sem_s.at[k])
    for k in range(NIDX):
        pltpu.async_copy(dst_hbm.at[pl.ds(ebase + k * CH, CH)], dsts[k],
                         sem_d.at[k])
    _fill(vals_0, CH, 0.0)
    for off in range(0, SLICE, CH):
        n = min(CH, SLICE - off)
        pltpu.sync_copy(vals_0.at[pl.ds(0, n)],
                        acc_sp.at[pl.ds(nb + off, n)])
    plsc.subcore_barrier()
    pltpu.make_async_copy(u_hbm, u_loc, sem_u.at[0]).wait()

    def _wait_src(k):
        pltpu.make_async_copy(
            src_hbm.at[pl.ds(ebase, CH)], srcs[k % 2], sem_s.at[k % 2]).wait()

    def _wait_dst(k):
        pltpu.make_async_copy(
            dst_hbm.at[pl.ds(ebase, CH)], dsts[k % NIDX],
            sem_d.at[k % NIDX]).wait()

    def _wait_scatter(k):
        pltpu.make_async_copy(
            vals[k % 2], acc_sp.at[dsts[k % NIDX]], sem_a.at[k % 2]).wait()

    def _gather(k):
        sref = srcs[k % 2]
        vref = vals[k % 2]

        def body(i, carry):
            d = pl.ds(i * LANES, LANES)
            vref[d] = plsc.load_gather(
                u_loc, [sref[d]], mask=jnp.full((LANES,), True))
            return carry

        lax.fori_loop(0, CH // LANES, body, 0)

    for k in range(NCH):
        if k >= 2:
            _wait_scatter(k - 2)
            if k + 1 < NCH:
                pltpu.async_copy(
                    dst_hbm.at[pl.ds(ebase + (k + 1) * CH, CH)],
                    dsts[(k + 1) % NIDX], sem_d.at[(k + 1) % NIDX])
        _wait_src(k)
        _gather(k)
        if k + 2 < NCH:
            pltpu.async_copy(src_hbm.at[pl.ds(ebase + (k + 2) * CH, CH)],
                             srcs[k % 2], sem_s.at[k % 2])
        _wait_dst(k)
        pltpu.async_copy(vals[k % 2], acc_sp.at[dsts[k % NIDX]],
                         sem_a.at[k % 2], add=True)

    _wait_scatter(NCH - 2)
    _wait_scatter(NCH - 1)
    plsc.subcore_barrier()
    for off in range(0, SLICE, CH):
        n = min(CH, SLICE - off)
        pltpu.sync_copy(acc_sp.at[pl.ds(nb + off, n)], vals_0.at[pl.ds(0, n)])
        pltpu.sync_copy(vals_0.at[pl.ds(0, n)],
                        out_hbm.at[pl.ds(c * NP + nb + off, n)])


def kernel(x, edge_index, Ws):
    del Ws
    src = edge_index[0]
    dst = edge_index[1]
    xcol = x[:, 3].astype(jnp.float32)
    xp = jnp.zeros((NP,), jnp.float32).at[:NN].set(xcol)

    degp = _deg_kernel(dst)
    deg = degp[:NP] + degp[NP:]
    dis = jnp.where(deg > 0, lax.rsqrt(jnp.maximum(deg, 1e-12)), 0.0)
    inv = dis * dis

    u = dis * xp
    for blk in range(NBLK):
        p = _spmv_kernel(u, src, dst)
        scale = dis if blk == NBLK - 1 else inv
        u = scale * (p[:NP] + p[NP:])
    return u[:NN].reshape(NN, 1, 1)
